# baseline (speedup 1.0000x reference)
import functools

import jax
import jax.numpy as jnp
from jax import lax
from jax.experimental import pallas as pl
from jax.experimental.pallas import tpu as pltpu
from jax.experimental.pallas import tpu_sc as plsc

GUMBEL_BETA = 0.01
EULER_GAMMA = 0.5772156649015329
EMB = 128
BM = 1024
CH = 32


def _sc_gather(h, t, r, emin, edel, relcat):
    info = plsc.get_sparse_core_info()
    nc, ns = info.num_cores, info.num_subcores
    nw = nc * ns
    batch = h.shape[0]
    bpw = batch // nw
    f32 = jnp.float32
    mesh = plsc.VectorSubcoreMesh(core_axis_name="c", subcore_axis_name="s")

    @functools.partial(
        pl.kernel,
        out_type=[
            jax.ShapeDtypeStruct((batch, EMB), f32),
            jax.ShapeDtypeStruct((batch, EMB), f32),
            jax.ShapeDtypeStruct((batch, EMB), f32),
            jax.ShapeDtypeStruct((batch, EMB), f32),
            jax.ShapeDtypeStruct((batch, 4 * EMB), f32),
        ],
        mesh=mesh,
        scratch_types=[
            pltpu.VMEM((bpw,), jnp.int32),
            pltpu.VMEM((bpw,), jnp.int32),
            pltpu.VMEM((bpw,), jnp.int32),
            pltpu.VMEM((2, CH, EMB), f32),
            pltpu.VMEM((2, CH, EMB), f32),
            pltpu.VMEM((2, CH, EMB), f32),
            pltpu.VMEM((2, CH, EMB), f32),
            pltpu.VMEM((2, CH, 4 * EMB), f32),
            pltpu.SemaphoreType.DMA,
            pltpu.SemaphoreType.DMA,
            pltpu.SemaphoreType.DMA,
            pltpu.SemaphoreType.DMA,
        ],
    )
    def gather_kernel(h_hbm, t_hbm, r_hbm, emin_hbm, edel_hbm, rel_hbm,
                      o_mh, o_dh, o_mt, o_dt, o_gr,
                      idx_h, idx_t, idx_r, b_mh, b_dh, b_mt, b_dt, b_gr,
                      gsem0, gsem1, wsem0, wsem1):
        wid = lax.axis_index("s") * nc + lax.axis_index("c")
        base = wid * bpw
        pltpu.sync_copy(h_hbm.at[pl.ds(base, bpw)], idx_h)
        pltpu.sync_copy(t_hbm.at[pl.ds(base, bpw)], idx_t)
        pltpu.sync_copy(r_hbm.at[pl.ds(base, bpw)], idx_r)

        gsems = (gsem0, gsem1)
        wsems = (wsem0, wsem1)

        def run_pipeline(tbls, idxs, bufs, outs, ch, nchunks):
            def fire_gathers(c, s):
                off = c * ch
                for tbl, idx, buf in zip(tbls, idxs, bufs):
                    pltpu.async_copy(tbl.at[idx.at[pl.ds(off, ch)]],
                                     buf.at[s], gsems[s])

            def wait_gathers(s):
                for tbl, idx, buf in zip(tbls, idxs, bufs):
                    pltpu.make_async_copy(tbl.at[idx.at[pl.ds(0, ch)]],
                                          buf.at[s], gsems[s]).wait()

            def fire_writes(c, s):
                off = c * ch
                for buf, out in zip(bufs, outs):
                    pltpu.async_copy(buf.at[s], out.at[pl.ds(base + off, ch)],
                                     wsems[s])

            def wait_writes(s):
                for buf, out in zip(bufs, outs):
                    pltpu.make_async_copy(buf.at[s], out.at[pl.ds(base, ch)],
                                          wsems[s]).wait()

            fire_gathers(0, 0)

            def pair(k, carry):
                c0 = 2 * k

                @pl.when(k > 0)
                def _():
                    wait_writes(1)

                fire_gathers(c0 + 1, 1)
                wait_gathers(0)
                fire_writes(c0, 0)
                wait_gathers(1)
                fire_writes(c0 + 1, 1)

                @pl.when(k < nchunks // 2 - 1)
                def _():
                    wait_writes(0)
                    fire_gathers(c0 + 2, 0)

                return carry

            lax.fori_loop(0, nchunks // 2, pair, 0)
            wait_writes(0)
            wait_writes(1)

        run_pipeline((emin_hbm, edel_hbm, emin_hbm, edel_hbm, rel_hbm),
                     (idx_h, idx_h, idx_t, idx_t, idx_r),
                     (b_mh, b_dh, b_mt, b_dt, b_gr),
                     (o_mh, o_dh, o_mt, o_dt, o_gr),
                     CH, bpw // CH)

    return gather_kernel(h, t, r, emin, edel, relcat)


def _log1pexp(x):
    return jnp.log1p(jnp.exp(x))


def _logaddexp(a, b):
    m = jnp.maximum(a, b)
    return m + _log1pexp(-jnp.abs(a - b))


def _softplus(x):
    return jnp.maximum(x, 0.0) + _log1pexp(-jnp.abs(x))


def _log_volume(delta):
    eps = jnp.finfo(jnp.float32).tiny
    sp = _softplus(delta - 2.0 * EULER_GAMMA * GUMBEL_BETA)
    return jnp.sum(jnp.log(jnp.maximum(sp, eps)), axis=-1, keepdims=True)


def _math_body(mh_ref, dh_ref, mt_ref, dt_ref, gr_ref, out_ref):
    gr = gr_ref[...]
    min_h = mh_ref[...]
    max_h = min_h + jnp.exp(dh_ref[...])
    delta_h = max_h - min_h
    trans_h = gr[:, 0:EMB]
    scale_h = jnp.maximum(gr[:, EMB:2 * EMB], 0.0)
    min_h = min_h + trans_h
    delta_h = delta_h * scale_h
    max_h = min_h + delta_h

    min_t = mt_ref[...]
    max_t = min_t + jnp.exp(dt_ref[...])
    delta_t = max_t - min_t
    trans_t = gr[:, 2 * EMB:3 * EMB]
    scale_t = jnp.maximum(gr[:, 3 * EMB:], 0.0)
    min_t = min_t + trans_t
    delta_t = delta_t * scale_t
    max_t = min_t + delta_t

    b = GUMBEL_BETA
    int_min = b * _logaddexp(min_h / b, min_t / b)
    int_min = jnp.maximum(int_min, jnp.maximum(min_h, min_t))
    int_max = -b * _logaddexp(-max_h / b, -max_t / b)
    int_max = jnp.minimum(int_max, jnp.minimum(max_h, max_t))

    li = _log_volume(int_max - int_min)
    lt = _log_volume(delta_t)
    out_ref[...] = jnp.exp(li - lt)


def _tc_math(mh, dh, mt, dt, gr):
    batch = mh.shape[0]
    grid = batch // BM
    out = pl.pallas_call(
        _math_body,
        grid=(grid,),
        in_specs=[
            pl.BlockSpec((BM, EMB), lambda i: (i, 0)),
            pl.BlockSpec((BM, EMB), lambda i: (i, 0)),
            pl.BlockSpec((BM, EMB), lambda i: (i, 0)),
            pl.BlockSpec((BM, EMB), lambda i: (i, 0)),
            pl.BlockSpec((BM, 4 * EMB), lambda i: (i, 0)),
        ],
        out_specs=pl.BlockSpec((BM, 1), lambda i: (i, 0)),
        out_shape=jax.ShapeDtypeStruct((batch, 1), jnp.float32),
    )(mh, dh, mt, dt, gr)
    return out[:, 0]


def kernel(ids, min_embedding, delta_embedding, rel_trans_for_head,
           rel_scale_for_head, rel_trans_for_tail, rel_scale_for_tail):
    batch = ids.shape[0]
    h = ids[:, 0]
    r = ids[:, 1]
    t = ids[:, 2]
    relcat = jnp.concatenate([rel_trans_for_head, rel_scale_for_head,
                              rel_trans_for_tail, rel_scale_for_tail], axis=1)

    g = _sc_gather(h, t, r, min_embedding, delta_embedding, relcat)
    return _tc_math(*g)


# speedup vs baseline: 1.0203x; 1.0203x over previous
import functools

import jax
import jax.numpy as jnp
from jax import lax
from jax.experimental import pallas as pl
from jax.experimental.pallas import tpu as pltpu
from jax.experimental.pallas import tpu_sc as plsc

GUMBEL_BETA = 0.01
EULER_GAMMA = 0.5772156649015329
EMB = 128
BM = 2048
CH = 32


def _sc_gather(h, t, r, emin, edel, relcat):
    info = plsc.get_sparse_core_info()
    nc, ns = info.num_cores, info.num_subcores
    nw = nc * ns
    batch = h.shape[0]
    bpw = batch // nw
    f32 = jnp.float32
    mesh = plsc.VectorSubcoreMesh(core_axis_name="c", subcore_axis_name="s")

    @functools.partial(
        pl.kernel,
        out_type=[
            jax.ShapeDtypeStruct((batch, EMB), f32),
            jax.ShapeDtypeStruct((batch, EMB), f32),
            jax.ShapeDtypeStruct((batch, EMB), f32),
            jax.ShapeDtypeStruct((batch, EMB), f32),
            jax.ShapeDtypeStruct((batch, 4 * EMB), f32),
        ],
        mesh=mesh,
        scratch_types=[
            pltpu.VMEM((bpw,), jnp.int32),
            pltpu.VMEM((bpw,), jnp.int32),
            pltpu.VMEM((bpw,), jnp.int32),
            pltpu.VMEM((2, CH, EMB), f32),
            pltpu.VMEM((2, CH, EMB), f32),
            pltpu.VMEM((2, CH, EMB), f32),
            pltpu.VMEM((2, CH, EMB), f32),
            pltpu.VMEM((2, CH, 4 * EMB), f32),
            pltpu.SemaphoreType.DMA,
            pltpu.SemaphoreType.DMA,
            pltpu.SemaphoreType.DMA,
            pltpu.SemaphoreType.DMA,
        ],
    )
    def gather_kernel(h_hbm, t_hbm, r_hbm, emin_hbm, edel_hbm, rel_hbm,
                      o_mh, o_dh, o_mt, o_dt, o_gr,
                      idx_h, idx_t, idx_r, b_mh, b_dh, b_mt, b_dt, b_gr,
                      gsem0, gsem1, wsem0, wsem1):
        wid = lax.axis_index("s") * nc + lax.axis_index("c")
        base = wid * bpw
        pltpu.sync_copy(h_hbm.at[pl.ds(base, bpw)], idx_h)
        pltpu.sync_copy(t_hbm.at[pl.ds(base, bpw)], idx_t)
        pltpu.sync_copy(r_hbm.at[pl.ds(base, bpw)], idx_r)

        gsems = (gsem0, gsem1)
        wsems = (wsem0, wsem1)

        def run_pipeline(tbls, idxs, bufs, outs, ch, nchunks):
            def fire_gathers(c, s):
                off = c * ch
                for tbl, idx, buf in zip(tbls, idxs, bufs):
                    pltpu.async_copy(tbl.at[idx.at[pl.ds(off, ch)]],
                                     buf.at[s], gsems[s])

            def wait_gathers(s):
                for tbl, idx, buf in zip(tbls, idxs, bufs):
                    pltpu.make_async_copy(tbl.at[idx.at[pl.ds(0, ch)]],
                                          buf.at[s], gsems[s]).wait()

            def fire_writes(c, s):
                off = c * ch
                for buf, out in zip(bufs, outs):
                    pltpu.async_copy(buf.at[s], out.at[pl.ds(base + off, ch)],
                                     wsems[s])

            def wait_writes(s):
                for buf, out in zip(bufs, outs):
                    pltpu.make_async_copy(buf.at[s], out.at[pl.ds(base, ch)],
                                          wsems[s]).wait()

            fire_gathers(0, 0)

            def pair(k, carry):
                c0 = 2 * k

                @pl.when(k > 0)
                def _():
                    wait_writes(1)

                fire_gathers(c0 + 1, 1)
                wait_gathers(0)
                fire_writes(c0, 0)
                wait_gathers(1)
                fire_writes(c0 + 1, 1)

                @pl.when(k < nchunks // 2 - 1)
                def _():
                    wait_writes(0)
                    fire_gathers(c0 + 2, 0)

                return carry

            lax.fori_loop(0, nchunks // 2, pair, 0)
            wait_writes(0)
            wait_writes(1)

        run_pipeline((emin_hbm, edel_hbm, emin_hbm, edel_hbm, rel_hbm),
                     (idx_h, idx_h, idx_t, idx_t, idx_r),
                     (b_mh, b_dh, b_mt, b_dt, b_gr),
                     (o_mh, o_dh, o_mt, o_dt, o_gr),
                     CH, bpw // CH)

    return gather_kernel(h, t, r, emin, edel, relcat)


def _log1pexp(x):
    return jnp.log1p(jnp.exp(x))


def _logaddexp(a, b):
    m = jnp.maximum(a, b)
    return m + _log1pexp(-jnp.abs(a - b))


def _softplus(x):
    return jnp.maximum(x, 0.0) + _log1pexp(-jnp.abs(x))


def _log_volume(delta):
    eps = jnp.finfo(jnp.float32).tiny
    sp = _softplus(delta - 2.0 * EULER_GAMMA * GUMBEL_BETA)
    return jnp.sum(jnp.log(jnp.maximum(sp, eps)), axis=-1, keepdims=True)


def _math_body(mh_ref, dh_ref, mt_ref, dt_ref, gr_ref, out_ref):
    gr = gr_ref[...]
    min_h = mh_ref[...]
    max_h = min_h + jnp.exp(dh_ref[...])
    delta_h = max_h - min_h
    trans_h = gr[:, 0:EMB]
    scale_h = jnp.maximum(gr[:, EMB:2 * EMB], 0.0)
    min_h = min_h + trans_h
    delta_h = delta_h * scale_h
    max_h = min_h + delta_h

    min_t = mt_ref[...]
    max_t = min_t + jnp.exp(dt_ref[...])
    delta_t = max_t - min_t
    trans_t = gr[:, 2 * EMB:3 * EMB]
    scale_t = jnp.maximum(gr[:, 3 * EMB:], 0.0)
    min_t = min_t + trans_t
    delta_t = delta_t * scale_t
    max_t = min_t + delta_t

    b = GUMBEL_BETA
    int_min = b * _logaddexp(min_h / b, min_t / b)
    int_min = jnp.maximum(int_min, jnp.maximum(min_h, min_t))
    int_max = -b * _logaddexp(-max_h / b, -max_t / b)
    int_max = jnp.minimum(int_max, jnp.minimum(max_h, max_t))

    li = _log_volume(int_max - int_min)
    lt = _log_volume(delta_t)
    out_ref[...] = jnp.exp(li - lt)


def _tc_math(mh, dh, mt, dt, gr):
    batch = mh.shape[0]
    grid = batch // BM
    out = pl.pallas_call(
        _math_body,
        grid=(grid,),
        in_specs=[
            pl.BlockSpec((BM, EMB), lambda i: (i, 0)),
            pl.BlockSpec((BM, EMB), lambda i: (i, 0)),
            pl.BlockSpec((BM, EMB), lambda i: (i, 0)),
            pl.BlockSpec((BM, EMB), lambda i: (i, 0)),
            pl.BlockSpec((BM, 4 * EMB), lambda i: (i, 0)),
        ],
        out_specs=pl.BlockSpec((BM, 1), lambda i: (i, 0)),
        out_shape=jax.ShapeDtypeStruct((batch, 1), jnp.float32),
    )(mh, dh, mt, dt, gr)
    return out[:, 0]


def kernel(ids, min_embedding, delta_embedding, rel_trans_for_head,
           rel_scale_for_head, rel_trans_for_tail, rel_scale_for_tail):
    batch = ids.shape[0]
    h = ids[:, 0]
    r = ids[:, 1]
    t = ids[:, 2]
    relcat = jnp.concatenate([rel_trans_for_head, rel_scale_for_head,
                              rel_trans_for_tail, rel_scale_for_tail], axis=1)

    g = _sc_gather(h, t, r, min_embedding, delta_embedding, relcat)
    return _tc_math(*g)
